# baseline (speedup 1.0000x reference)
import functools

import jax
import jax.numpy as jnp
from jax import lax
from jax.experimental import pallas as pl
from jax.experimental.pallas import tpu as pltpu
from jax.experimental.pallas import tpu_sc as plsc

_N = 10000
_E = 320000
_D = 128
_NP = 10240
_NW = 32
_CHUNK = 128
_CPT = 79
_CPT_H = 80
_WCH = 16
_NWIN = 5

_mesh = plsc.VectorSubcoreMesh(core_axis_name="c", subcore_axis_name="s")


@functools.partial(
    pl.kernel,
    mesh=_mesh,
    out_type=jax.ShapeDtypeStruct((2, _NP, _D), jnp.float32),
    scratch_types=[
        pltpu.VMEM((_CPT_H, _CHUNK), jnp.int32),
        pltpu.VMEM((_CHUNK, _D), jnp.float32),
        pltpu.VMEM((64, _D), jnp.float32),
        pltpu.VMEM_SHARED((_NP, _D), jnp.float32),
        pltpu.SemaphoreType.DMA,
    ],
)
def _deg_kernel(dst_hbm, ones_hbm, out_hbm, idx_v, one_v, zer_v, acc_s,
                ssem):
    cid = lax.axis_index("c")
    sid = lax.axis_index("s")
    wid = sid * 2 + cid
    pltpu.sync_copy(dst_hbm.at[wid], idx_v)
    pltpu.sync_copy(ones_hbm, one_v)

    def _zero_buf(i, carry):
        zer_v[i // 8, pl.ds((i % 8) * 16, 16)] = jnp.zeros((16,), jnp.float32)
        return carry

    lax.fori_loop(0, 64 * 8, _zero_buf, 0)

    rows_per_sub = _NP // 16
    base = sid * rows_per_sub

    def _zero_acc(k, carry):
        pltpu.sync_copy(zer_v, acc_s.at[pl.ds(base + k * 64, 64)])
        return carry

    lax.fori_loop(0, rows_per_sub // 64, _zero_acc, 0)
    plsc.subcore_barrier()

    def _edges(g, carry):
        for k in range(8):
            @pl.when(8 * g + k < _CPT)
            def _():
                pltpu.async_copy(one_v, acc_s.at[idx_v.at[8 * g + k]],
                                 ssem, add=True)
        for k in range(8):
            @pl.when(8 * g + k < _CPT)
            def _():
                pltpu.make_async_copy(one_v, acc_s.at[pl.ds(0, _CHUNK)],
                                      ssem).wait()
        return carry

    lax.fori_loop(0, (_CPT + 7) // 8, _edges, 0)
    plsc.subcore_barrier()
    pltpu.sync_copy(acc_s.at[pl.ds(base, rows_per_sub)],
                    out_hbm.at[cid, pl.ds(base, rows_per_sub)])


@functools.partial(
    pl.kernel,
    mesh=_mesh,
    out_type=jax.ShapeDtypeStruct((2, _NP, _D), jnp.float32),
    scratch_types=[
        pltpu.VMEM((2, _WCH, _CHUNK), jnp.int32),
        pltpu.VMEM((2, _WCH, _CHUNK), jnp.int32),
        pltpu.VMEM((2, _CHUNK, _D), jnp.float32),
        pltpu.VMEM((32, _D), jnp.float32),
        pltpu.VMEM_SHARED((_NP, _D), jnp.float32),
        pltpu.SemaphoreType.DMA,
        pltpu.SemaphoreType.DMA,
        pltpu.SemaphoreType.DMA,
        pltpu.SemaphoreType.DMA,
        pltpu.SemaphoreType.DMA,
    ],
)
def _edge_sum_kernel(z_hbm, src_hbm, dst_hbm, out_hbm,
                     srcw, dstw, rows_v, zer_v, acc_s,
                     gsem0, gsem1, ssem0, ssem1, isem):
    cid = lax.axis_index("c")
    sid = lax.axis_index("s")
    wid = sid * 2 + cid

    def _zero_buf(i, carry):
        zer_v[i // 8, pl.ds((i % 8) * 16, 16)] = jnp.zeros((16,), jnp.float32)
        return carry

    lax.fori_loop(0, 32 * 8, _zero_buf, 0)

    rows_per_sub = _NP // 16
    base = sid * rows_per_sub

    def _zero_acc(k, carry):
        pltpu.sync_copy(zer_v, acc_s.at[pl.ds(base + k * 32, 32)])
        return carry

    lax.fori_loop(0, rows_per_sub // 32, _zero_acc, 0)

    pltpu.sync_copy(src_hbm.at[wid, pl.ds(0, _WCH)], srcw.at[0])
    pltpu.sync_copy(dst_hbm.at[wid, pl.ds(0, _WCH)], dstw.at[0])
    plsc.subcore_barrier()

    pltpu.async_copy(z_hbm.at[srcw.at[0, 0]], rows_v.at[0], gsem0)
    pltpu.async_copy(z_hbm.at[srcw.at[0, 1]], rows_v.at[1], gsem1)

    def _win_src(w):
        return src_hbm.at[wid, pl.ds(w * _WCH, _WCH)]

    def _win_dst(w):
        return dst_hbm.at[wid, pl.ds(w * _WCH, _WCH)]

    def _scat_drain(sem):
        return pltpu.make_async_copy(rows_v.at[0],
                                     acc_s.at[pl.ds(0, _CHUNK)], sem)

    def _pipe(g, carry):
        j0 = 2 * g
        w = j0 // _WCH
        h = lax.rem(w, 2)
        s0 = j0 - w * _WCH
        gpos = lax.rem(g, _WCH // 2)

        @pl.when((gpos == 0) & (w < _NWIN - 1))
        def _():
            pltpu.async_copy(_win_src(w + 1), srcw.at[1 - h], isem)
            pltpu.async_copy(_win_dst(w + 1), dstw.at[1 - h], isem)

        pltpu.make_async_copy(z_hbm.at[srcw.at[h, s0]], rows_v.at[0],
                              gsem0).wait()
        pltpu.async_copy(rows_v.at[0], acc_s.at[dstw.at[h, s0]], ssem0,
                         add=True)
        pltpu.make_async_copy(z_hbm.at[srcw.at[h, s0 + 1]], rows_v.at[1],
                              gsem1).wait()
        pltpu.async_copy(rows_v.at[1], acc_s.at[dstw.at[h, s0 + 1]], ssem1,
                         add=True)

        @pl.when(gpos == (_WCH // 2) - 1)
        def _():
            pltpu.make_async_copy(_win_src(w + 1), srcw.at[1 - h],
                                  isem).wait()
            pltpu.make_async_copy(_win_dst(w + 1), dstw.at[1 - h],
                                  isem).wait()

        j2 = j0 + 2
        w2 = j2 // _WCH
        h2 = lax.rem(w2, 2)
        _scat_drain(ssem0).wait()
        pltpu.async_copy(
            z_hbm.at[srcw.at[h2, j2 - w2 * _WCH]], rows_v.at[0], gsem0)
        _scat_drain(ssem1).wait()

        @pl.when(j2 + 1 < _CPT)
        def _():
            pltpu.async_copy(
                z_hbm.at[srcw.at[h2, j2 + 1 - w2 * _WCH]],
                rows_v.at[1], gsem1)

        return carry

    lax.fori_loop(0, _CPT // 2, _pipe, 0)
    lastw = (_CPT - 1) // _WCH
    pltpu.make_async_copy(
        z_hbm.at[srcw.at[lastw % 2, (_CPT - 1) - lastw * _WCH]],
        rows_v.at[0], gsem0).wait()
    pltpu.async_copy(
        rows_v.at[0], acc_s.at[dstw.at[lastw % 2,
                                       (_CPT - 1) - lastw * _WCH]],
        ssem0, add=True)
    _scat_drain(ssem0).wait()
    plsc.subcore_barrier()
    pltpu.sync_copy(acc_s.at[pl.ds(base, rows_per_sub)],
                    out_hbm.at[cid, pl.ds(base, rows_per_sub)])


_BM = 256
_GRID = _NP // _BM


def _k1_body(x_ref, we_ref, be_ref, w1_ref, degp_ref, z_ref, y_ref):
    deg = (degp_ref[0, :, 0:1] + degp_ref[1, :, 0:1]) + 1.0
    yv = lax.rsqrt(deg)
    h = jnp.dot(x_ref[...], we_ref[...],
                preferred_element_type=jnp.float32) + be_ref[...]
    z_ref[...] = yv * jnp.dot(h, w1_ref[...],
                              preferred_element_type=jnp.float32)
    y_ref[...] = yv


def _k2_body(accp_ref, z1_ref, y_ref, b1_ref, w2_ref, z2_ref):
    yv = y_ref[...]
    acc = accp_ref[0] + accp_ref[1] + z1_ref[...]
    h = jnp.maximum(yv * acc + b1_ref[...], 0.0)
    z2_ref[...] = yv * jnp.dot(h, w2_ref[...],
                               preferred_element_type=jnp.float32)


def _k3_body(accp_ref, z2_ref, y_ref, b2_ref, wd_ref, bd_ref, o_ref):
    yv = y_ref[...]
    acc = accp_ref[0] + accp_ref[1] + z2_ref[...]
    h = jnp.maximum(yv * acc + b2_ref[...], 0.0)
    logits = jnp.dot(h, wd_ref[...],
                     preferred_element_type=jnp.float32) + bd_ref[...]
    m = jnp.max(logits, axis=1, keepdims=True)
    lse = jnp.log(jnp.sum(jnp.exp(logits - m), axis=1, keepdims=True)) + m
    o_ref[...] = logits - lse


def _row_spec(width):
    return pl.BlockSpec((_BM, width), lambda i: (i, 0))


_full_w = pl.BlockSpec((_D, _D), lambda i: (0, 0))
_full_b = pl.BlockSpec((1, _D), lambda i: (0, 0))

_k1 = pl.pallas_call(
    _k1_body,
    grid=(_GRID,),
    in_specs=[
        _row_spec(_D), _full_w, _full_b, _full_w,
        pl.BlockSpec((2, _BM, _D), lambda i: (0, i, 0)),
    ],
    out_specs=(_row_spec(_D), _row_spec(1)),
    out_shape=(
        jax.ShapeDtypeStruct((_NP, _D), jnp.float32),
        jax.ShapeDtypeStruct((_NP, 1), jnp.float32),
    ),
)

_k2 = pl.pallas_call(
    _k2_body,
    grid=(_GRID,),
    in_specs=[
        pl.BlockSpec((2, _BM, _D), lambda i: (0, i, 0)),
        _row_spec(_D), _row_spec(1), _full_b, _full_w,
    ],
    out_specs=_row_spec(_D),
    out_shape=jax.ShapeDtypeStruct((_NP, _D), jnp.float32),
)

_k3 = pl.pallas_call(
    _k3_body,
    grid=(_GRID,),
    in_specs=[
        pl.BlockSpec((2, _BM, _D), lambda i: (0, i, 0)),
        _row_spec(_D), _row_spec(1), _full_b, _full_w, _full_b,
    ],
    out_specs=_row_spec(_D),
    out_shape=jax.ShapeDtypeStruct((_NP, _D), jnp.float32),
)


def kernel(X, edge_index, W_enc, b_enc, W_gcn1, b_gcn1, W_gcn2, b_gcn2,
           W_dec, b_dec):
    Xp = jnp.pad(X, ((0, _NP - _N), (0, 0)))
    fill = jnp.full((_NW * _CPT * _CHUNK - _E,), _NP - 1, jnp.int32)
    pad_chunk = jnp.full((_NW, 1, _CHUNK), _NP - 1, jnp.int32)

    def _chunked(idx):
        body = jnp.concatenate([idx, fill]).reshape(_NW, _CPT, _CHUNK)
        return jnp.concatenate([body, pad_chunk], axis=1)

    srcs = _chunked(edge_index[0])
    dsts = _chunked(edge_index[1])

    ones_rows = jnp.ones((_CHUNK, _D), jnp.float32)
    deg_parts = _deg_kernel(dsts, ones_rows)
    z1, y = _k1(Xp, W_enc.T, b_enc.reshape(1, _D), W_gcn1.T, deg_parts)
    acc1 = _edge_sum_kernel(z1, srcs, dsts)
    z2 = _k2(acc1, z1, y, b_gcn1.reshape(1, _D), W_gcn2.T)
    acc2 = _edge_sum_kernel(z2, srcs, dsts)
    out = _k3(acc2, z2, y, b_gcn2.reshape(1, _D), W_dec.T, b_dec.reshape(1, _D))
    return out[:_N]


# speedup vs baseline: 1.0760x; 1.0760x over previous
import functools

import jax
import jax.numpy as jnp
from jax import lax
from jax.experimental import pallas as pl
from jax.experimental.pallas import tpu as pltpu
from jax.experimental.pallas import tpu_sc as plsc

_N = 10000
_E = 320000
_D = 128
_NP = 10240
_NW = 32
_CHUNK = 128
_CPT = 79
_CPT_H = 80
_WCH = 16
_NWIN = 5

_mesh = plsc.VectorSubcoreMesh(core_axis_name="c", subcore_axis_name="s")


@functools.partial(
    pl.kernel,
    mesh=_mesh,
    out_type=jax.ShapeDtypeStruct((2, _NP, _D), jnp.float32),
    scratch_types=[
        pltpu.VMEM((_CPT_H, _CHUNK), jnp.int32),
        pltpu.VMEM((_CHUNK, _D), jnp.float32),
        pltpu.VMEM((64, _D), jnp.float32),
        pltpu.VMEM_SHARED((_NP, _D), jnp.float32),
        pltpu.SemaphoreType.DMA,
    ],
)
def _deg_kernel(dst_hbm, ones_hbm, out_hbm, idx_v, one_v, zer_v, acc_s,
                ssem):
    cid = lax.axis_index("c")
    sid = lax.axis_index("s")
    wid = sid * 2 + cid
    pltpu.sync_copy(dst_hbm.at[wid], idx_v)
    pltpu.sync_copy(ones_hbm, one_v)

    def _zero_buf(i, carry):
        zer_v[i // 8, pl.ds((i % 8) * 16, 16)] = jnp.zeros((16,), jnp.float32)
        return carry

    lax.fori_loop(0, 64 * 8, _zero_buf, 0)

    rows_per_sub = _NP // 16
    base = sid * rows_per_sub

    def _zero_acc(k, carry):
        pltpu.sync_copy(zer_v, acc_s.at[pl.ds(base + k * 64, 64)])
        return carry

    lax.fori_loop(0, rows_per_sub // 64, _zero_acc, 0)
    plsc.subcore_barrier()

    def _edges(g, carry):
        for k in range(8):
            @pl.when(8 * g + k < _CPT)
            def _():
                pltpu.async_copy(one_v, acc_s.at[idx_v.at[8 * g + k]],
                                 ssem, add=True)
        for k in range(8):
            @pl.when(8 * g + k < _CPT)
            def _():
                pltpu.make_async_copy(one_v, acc_s.at[pl.ds(0, _CHUNK)],
                                      ssem).wait()
        return carry

    lax.fori_loop(0, (_CPT + 7) // 8, _edges, 0)
    plsc.subcore_barrier()
    pltpu.sync_copy(acc_s.at[pl.ds(base, rows_per_sub)],
                    out_hbm.at[cid, pl.ds(base, rows_per_sub)])


@functools.partial(
    pl.kernel,
    mesh=_mesh,
    out_type=jax.ShapeDtypeStruct((2, _NP, _D), jnp.float32),
    scratch_types=[
        pltpu.VMEM((2, _WCH, _CHUNK), jnp.int32),
        pltpu.VMEM((2, _WCH, _CHUNK), jnp.int32),
        pltpu.VMEM((2, _CHUNK, _D), jnp.float32),
        pltpu.VMEM((32, _D), jnp.float32),
        pltpu.VMEM_SHARED((_NP, _D), jnp.float32),
        pltpu.SemaphoreType.DMA,
        pltpu.SemaphoreType.DMA,
        pltpu.SemaphoreType.DMA,
    ],
)
def _edge_sum_kernel(z_hbm, src_hbm, dst_hbm, out_hbm,
                     srcw, dstw, rows_v, zer_v, acc_s, gsem0, gsem1, isem):
    cid = lax.axis_index("c")
    sid = lax.axis_index("s")
    wid = sid * 2 + cid

    def _zero_buf(i, carry):
        zer_v[i // 8, pl.ds((i % 8) * 16, 16)] = jnp.zeros((16,), jnp.float32)
        return carry

    lax.fori_loop(0, 32 * 8, _zero_buf, 0)

    rows_per_sub = _NP // 16
    base = sid * rows_per_sub

    def _zero_acc(k, carry):
        pltpu.sync_copy(zer_v, acc_s.at[pl.ds(base + k * 32, 32)])
        return carry

    lax.fori_loop(0, rows_per_sub // 32, _zero_acc, 0)

    pltpu.sync_copy(src_hbm.at[wid, pl.ds(0, _WCH)], srcw.at[0])
    pltpu.sync_copy(dst_hbm.at[wid, pl.ds(0, _WCH)], dstw.at[0])
    plsc.subcore_barrier()

    pltpu.async_copy(z_hbm.at[srcw.at[0, 0]], rows_v.at[0], gsem0)

    def _win_src(w):
        return src_hbm.at[wid, pl.ds(w * _WCH, _WCH)]

    def _win_dst(w):
        return dst_hbm.at[wid, pl.ds(w * _WCH, _WCH)]

    def _pipe(g, carry):
        j0 = 2 * g
        w = j0 // _WCH
        h = lax.rem(w, 2)
        s0 = j0 - w * _WCH
        gpos = lax.rem(g, _WCH // 2)

        @pl.when((gpos == 0) & (w < _NWIN - 1))
        def _():
            pltpu.async_copy(_win_src(w + 1), srcw.at[1 - h], isem)
            pltpu.async_copy(_win_dst(w + 1), dstw.at[1 - h], isem)

        pltpu.async_copy(z_hbm.at[srcw.at[h, s0 + 1]], rows_v.at[1],
                         gsem1)
        pltpu.make_async_copy(z_hbm.at[srcw.at[h, s0]], rows_v.at[0],
                              gsem0).wait()
        pltpu.sync_copy(rows_v.at[0], acc_s.at[dstw.at[h, s0]], add=True)

        @pl.when(gpos == (_WCH // 2) - 1)
        def _():
            pltpu.make_async_copy(_win_src(w + 1), srcw.at[1 - h],
                                  isem).wait()
            pltpu.make_async_copy(_win_dst(w + 1), dstw.at[1 - h],
                                  isem).wait()

        j2 = j0 + 2
        w2 = j2 // _WCH
        pltpu.async_copy(
            z_hbm.at[srcw.at[lax.rem(w2, 2), j2 - w2 * _WCH]],
            rows_v.at[0], gsem0)
        pltpu.make_async_copy(z_hbm.at[srcw.at[h, s0 + 1]], rows_v.at[1],
                              gsem1).wait()
        pltpu.sync_copy(rows_v.at[1], acc_s.at[dstw.at[h, s0 + 1]],
                        add=True)
        return carry

    lax.fori_loop(0, _CPT // 2, _pipe, 0)
    lastw = (_CPT - 1) // _WCH
    pltpu.make_async_copy(
        z_hbm.at[srcw.at[lastw % 2, (_CPT - 1) - lastw * _WCH]],
        rows_v.at[0], gsem0).wait()
    pltpu.sync_copy(rows_v.at[0],
                    acc_s.at[dstw.at[lastw % 2, (_CPT - 1) - lastw * _WCH]],
                    add=True)
    plsc.subcore_barrier()
    pltpu.sync_copy(acc_s.at[pl.ds(base, rows_per_sub)],
                    out_hbm.at[cid, pl.ds(base, rows_per_sub)])


_BM = 256
_GRID = _NP // _BM


def _k1_body(x_ref, we_ref, be_ref, w1_ref, degp_ref, z_ref, y_ref):
    deg = (degp_ref[0, :, 0:1] + degp_ref[1, :, 0:1]) + 1.0
    yv = lax.rsqrt(deg)
    h = jnp.dot(x_ref[...], we_ref[...],
                preferred_element_type=jnp.float32) + be_ref[...]
    z_ref[...] = yv * jnp.dot(h, w1_ref[...],
                              preferred_element_type=jnp.float32)
    y_ref[...] = yv


def _k2_body(accp_ref, z1_ref, y_ref, b1_ref, w2_ref, z2_ref):
    yv = y_ref[...]
    acc = accp_ref[0] + accp_ref[1] + z1_ref[...]
    h = jnp.maximum(yv * acc + b1_ref[...], 0.0)
    z2_ref[...] = yv * jnp.dot(h, w2_ref[...],
                               preferred_element_type=jnp.float32)


def _k3_body(accp_ref, z2_ref, y_ref, b2_ref, wd_ref, bd_ref, o_ref):
    yv = y_ref[...]
    acc = accp_ref[0] + accp_ref[1] + z2_ref[...]
    h = jnp.maximum(yv * acc + b2_ref[...], 0.0)
    logits = jnp.dot(h, wd_ref[...],
                     preferred_element_type=jnp.float32) + bd_ref[...]
    m = jnp.max(logits, axis=1, keepdims=True)
    lse = jnp.log(jnp.sum(jnp.exp(logits - m), axis=1, keepdims=True)) + m
    o_ref[...] = logits - lse


def _row_spec(width):
    return pl.BlockSpec((_BM, width), lambda i: (i, 0))


_full_w = pl.BlockSpec((_D, _D), lambda i: (0, 0))
_full_b = pl.BlockSpec((1, _D), lambda i: (0, 0))

_k1 = pl.pallas_call(
    _k1_body,
    grid=(_GRID,),
    in_specs=[
        _row_spec(_D), _full_w, _full_b, _full_w,
        pl.BlockSpec((2, _BM, _D), lambda i: (0, i, 0)),
    ],
    out_specs=(_row_spec(_D), _row_spec(1)),
    out_shape=(
        jax.ShapeDtypeStruct((_NP, _D), jnp.float32),
        jax.ShapeDtypeStruct((_NP, 1), jnp.float32),
    ),
)

_k2 = pl.pallas_call(
    _k2_body,
    grid=(_GRID,),
    in_specs=[
        pl.BlockSpec((2, _BM, _D), lambda i: (0, i, 0)),
        _row_spec(_D), _row_spec(1), _full_b, _full_w,
    ],
    out_specs=_row_spec(_D),
    out_shape=jax.ShapeDtypeStruct((_NP, _D), jnp.float32),
)

_k3 = pl.pallas_call(
    _k3_body,
    grid=(_GRID,),
    in_specs=[
        pl.BlockSpec((2, _BM, _D), lambda i: (0, i, 0)),
        _row_spec(_D), _row_spec(1), _full_b, _full_w, _full_b,
    ],
    out_specs=_row_spec(_D),
    out_shape=jax.ShapeDtypeStruct((_NP, _D), jnp.float32),
)


def kernel(X, edge_index, W_enc, b_enc, W_gcn1, b_gcn1, W_gcn2, b_gcn2,
           W_dec, b_dec):
    Xp = jnp.pad(X, ((0, _NP - _N), (0, 0)))
    fill = jnp.full((_NW * _CPT * _CHUNK - _E,), _NP - 1, jnp.int32)
    pad_chunk = jnp.full((_NW, 1, _CHUNK), _NP - 1, jnp.int32)

    def _chunked(idx):
        body = jnp.concatenate([idx, fill]).reshape(_NW, _CPT, _CHUNK)
        return jnp.concatenate([body, pad_chunk], axis=1)

    srcs = _chunked(edge_index[0])
    dsts = _chunked(edge_index[1])

    ones_rows = jnp.ones((_CHUNK, _D), jnp.float32)
    deg_parts = _deg_kernel(dsts, ones_rows)
    z1, y = _k1(Xp, W_enc.T, b_enc.reshape(1, _D), W_gcn1.T, deg_parts)
    acc1 = _edge_sum_kernel(z1, srcs, dsts)
    z2 = _k2(acc1, z1, y, b_gcn1.reshape(1, _D), W_gcn2.T)
    acc2 = _edge_sum_kernel(z2, srcs, dsts)
    out = _k3(acc2, z2, y, b_gcn2.reshape(1, _D), W_dec.T, b_dec.reshape(1, _D))
    return out[:_N]


# speedup vs baseline: 1.0812x; 1.0049x over previous
import functools

import jax
import jax.numpy as jnp
from jax import lax
from jax.experimental import pallas as pl
from jax.experimental.pallas import tpu as pltpu
from jax.experimental.pallas import tpu_sc as plsc

_N = 10000
_E = 320000
_D = 128
_NP = 10240
_NW = 32
_CHUNK = 128
_CPT = 79
_CPT_H = 80
_WCH = 16
_NWIN = 5

_mesh = plsc.VectorSubcoreMesh(core_axis_name="c", subcore_axis_name="s")


@functools.partial(
    pl.kernel,
    mesh=_mesh,
    out_type=jax.ShapeDtypeStruct((2, _NP, _D), jnp.float32),
    scratch_types=[
        pltpu.VMEM((_CPT_H, _CHUNK), jnp.int32),
        pltpu.VMEM((_CHUNK, _D), jnp.float32),
        pltpu.VMEM((64, _D), jnp.float32),
        pltpu.VMEM_SHARED((_NP, _D), jnp.float32),
        pltpu.SemaphoreType.DMA,
    ],
)
def _deg_kernel(dst_hbm, ones_hbm, out_hbm, idx_v, one_v, zer_v, acc_s,
                ssem):
    cid = lax.axis_index("c")
    sid = lax.axis_index("s")
    wid = sid * 2 + cid
    pltpu.sync_copy(dst_hbm.at[wid], idx_v)
    pltpu.sync_copy(ones_hbm, one_v)

    def _zero_buf(i, carry):
        zer_v[i // 8, pl.ds((i % 8) * 16, 16)] = jnp.zeros((16,), jnp.float32)
        return carry

    lax.fori_loop(0, 64 * 8, _zero_buf, 0)

    rows_per_sub = _NP // 16
    base = sid * rows_per_sub

    def _zero_acc(k, carry):
        pltpu.sync_copy(zer_v, acc_s.at[pl.ds(base + k * 64, 64)])
        return carry

    lax.fori_loop(0, rows_per_sub // 64, _zero_acc, 0)
    plsc.subcore_barrier()

    def _edges(g, carry):
        for k in range(8):
            @pl.when(8 * g + k < _CPT)
            def _():
                pltpu.async_copy(one_v, acc_s.at[idx_v.at[8 * g + k]],
                                 ssem, add=True)
        for k in range(8):
            @pl.when(8 * g + k < _CPT)
            def _():
                pltpu.make_async_copy(one_v, acc_s.at[pl.ds(0, _CHUNK)],
                                      ssem).wait()
        return carry

    lax.fori_loop(0, (_CPT + 7) // 8, _edges, 0)
    plsc.subcore_barrier()
    pltpu.sync_copy(acc_s.at[pl.ds(base, rows_per_sub)],
                    out_hbm.at[cid, pl.ds(base, rows_per_sub)])


@functools.partial(
    pl.kernel,
    mesh=_mesh,
    out_type=jax.ShapeDtypeStruct((2, _NP, _D), jnp.float32),
    scratch_types=[
        pltpu.VMEM((2, _WCH, _CHUNK), jnp.int32),
        pltpu.VMEM((2, _WCH, _CHUNK), jnp.int32),
        pltpu.VMEM((2, _CHUNK, _D), jnp.float32),
        pltpu.VMEM((32, _D), jnp.float32),
        pltpu.VMEM_SHARED((_NP, _D), jnp.float32),
        pltpu.SemaphoreType.DMA,
        pltpu.SemaphoreType.DMA,
        pltpu.SemaphoreType.DMA,
    ],
)
def _edge_sum_kernel(z_hbm, src_hbm, dst_hbm, out_hbm,
                     srcw, dstw, rows_v, zer_v, acc_s, gsem0, gsem1, isem):
    cid = lax.axis_index("c")
    sid = lax.axis_index("s")
    wid = sid * 2 + cid

    def _zero_buf(i, carry):
        zer_v[i // 8, pl.ds((i % 8) * 16, 16)] = jnp.zeros((16,), jnp.float32)
        return carry

    lax.fori_loop(0, 32 * 8, _zero_buf, 0)

    rows_per_sub = _NP // 16
    base = sid * rows_per_sub

    def _zero_acc(k, carry):
        pltpu.sync_copy(zer_v, acc_s.at[pl.ds(base + k * 32, 32)])
        return carry

    lax.fori_loop(0, rows_per_sub // 32, _zero_acc, 0)

    pltpu.sync_copy(src_hbm.at[wid, pl.ds(0, _WCH)], srcw.at[0])
    pltpu.sync_copy(dst_hbm.at[wid, pl.ds(0, _WCH)], dstw.at[0])
    plsc.subcore_barrier()

    pltpu.async_copy(z_hbm.at[srcw.at[0, 0]], rows_v.at[0], gsem0)

    def _win_src(w):
        return src_hbm.at[wid, pl.ds(w * _WCH, _WCH)]

    def _win_dst(w):
        return dst_hbm.at[wid, pl.ds(w * _WCH, _WCH)]

    def _pipe(g, carry):
        j0 = 2 * g
        w = j0 // _WCH
        h = lax.rem(w, 2)
        s0 = j0 - w * _WCH
        gpos = lax.rem(g, _WCH // 2)

        @pl.when((gpos == 0) & (w < _NWIN - 1))
        def _():
            pltpu.async_copy(_win_src(w + 1), srcw.at[1 - h], isem)
            pltpu.async_copy(_win_dst(w + 1), dstw.at[1 - h], isem)

        pltpu.async_copy(z_hbm.at[srcw.at[h, s0 + 1]], rows_v.at[1],
                         gsem1)
        pltpu.make_async_copy(z_hbm.at[srcw.at[h, s0]], rows_v.at[0],
                              gsem0).wait()
        pltpu.sync_copy(rows_v.at[0], acc_s.at[dstw.at[h, s0]], add=True)

        @pl.when(gpos == (_WCH // 2) - 1)
        def _():
            pltpu.make_async_copy(_win_src(w + 1), srcw.at[1 - h],
                                  isem).wait()
            pltpu.make_async_copy(_win_dst(w + 1), dstw.at[1 - h],
                                  isem).wait()

        j2 = j0 + 2
        w2 = j2 // _WCH
        pltpu.async_copy(
            z_hbm.at[srcw.at[lax.rem(w2, 2), j2 - w2 * _WCH]],
            rows_v.at[0], gsem0)
        pltpu.make_async_copy(z_hbm.at[srcw.at[h, s0 + 1]], rows_v.at[1],
                              gsem1).wait()
        pltpu.sync_copy(rows_v.at[1], acc_s.at[dstw.at[h, s0 + 1]],
                        add=True)
        return carry

    lax.fori_loop(0, _CPT // 2, _pipe, 0)
    lastw = (_CPT - 1) // _WCH
    pltpu.make_async_copy(
        z_hbm.at[srcw.at[lastw % 2, (_CPT - 1) - lastw * _WCH]],
        rows_v.at[0], gsem0).wait()
    pltpu.sync_copy(rows_v.at[0],
                    acc_s.at[dstw.at[lastw % 2, (_CPT - 1) - lastw * _WCH]],
                    add=True)
    plsc.subcore_barrier()
    pltpu.sync_copy(acc_s.at[pl.ds(base, rows_per_sub)],
                    out_hbm.at[cid, pl.ds(base, rows_per_sub)])


_BM = 256
_GRID = _NP // _BM


def _k0_body(x_ref, we_ref, be_ref, w1_ref, xw_ref):
    h = jnp.dot(x_ref[...], we_ref[...],
                preferred_element_type=jnp.float32) + be_ref[...]
    xw_ref[...] = jnp.dot(h, w1_ref[...],
                          preferred_element_type=jnp.float32)


def _k1_body(xw_ref, degp_ref, z_ref, y_ref):
    deg = (degp_ref[0, :, 0:1] + degp_ref[1, :, 0:1]) + 1.0
    yv = lax.rsqrt(deg)
    z_ref[...] = yv * xw_ref[...]
    y_ref[...] = yv


def _k2_body(accp_ref, z1_ref, y_ref, b1_ref, w2_ref, z2_ref):
    yv = y_ref[...]
    acc = accp_ref[0] + accp_ref[1] + z1_ref[...]
    h = jnp.maximum(yv * acc + b1_ref[...], 0.0)
    z2_ref[...] = yv * jnp.dot(h, w2_ref[...],
                               preferred_element_type=jnp.float32)


def _k3_body(accp_ref, z2_ref, y_ref, b2_ref, wd_ref, bd_ref, o_ref):
    yv = y_ref[...]
    acc = accp_ref[0] + accp_ref[1] + z2_ref[...]
    h = jnp.maximum(yv * acc + b2_ref[...], 0.0)
    logits = jnp.dot(h, wd_ref[...],
                     preferred_element_type=jnp.float32) + bd_ref[...]
    m = jnp.max(logits, axis=1, keepdims=True)
    lse = jnp.log(jnp.sum(jnp.exp(logits - m), axis=1, keepdims=True)) + m
    o_ref[...] = logits - lse


def _row_spec(width):
    return pl.BlockSpec((_BM, width), lambda i: (i, 0))


_full_w = pl.BlockSpec((_D, _D), lambda i: (0, 0))
_full_b = pl.BlockSpec((1, _D), lambda i: (0, 0))

_k0 = pl.pallas_call(
    _k0_body,
    grid=(_GRID,),
    in_specs=[_row_spec(_D), _full_w, _full_b, _full_w],
    out_specs=_row_spec(_D),
    out_shape=jax.ShapeDtypeStruct((_NP, _D), jnp.float32),
)

_k1 = pl.pallas_call(
    _k1_body,
    grid=(_GRID,),
    in_specs=[
        _row_spec(_D),
        pl.BlockSpec((2, _BM, _D), lambda i: (0, i, 0)),
    ],
    out_specs=(_row_spec(_D), _row_spec(1)),
    out_shape=(
        jax.ShapeDtypeStruct((_NP, _D), jnp.float32),
        jax.ShapeDtypeStruct((_NP, 1), jnp.float32),
    ),
)

_k2 = pl.pallas_call(
    _k2_body,
    grid=(_GRID,),
    in_specs=[
        pl.BlockSpec((2, _BM, _D), lambda i: (0, i, 0)),
        _row_spec(_D), _row_spec(1), _full_b, _full_w,
    ],
    out_specs=_row_spec(_D),
    out_shape=jax.ShapeDtypeStruct((_NP, _D), jnp.float32),
)

_k3 = pl.pallas_call(
    _k3_body,
    grid=(_GRID,),
    in_specs=[
        pl.BlockSpec((2, _BM, _D), lambda i: (0, i, 0)),
        _row_spec(_D), _row_spec(1), _full_b, _full_w, _full_b,
    ],
    out_specs=_row_spec(_D),
    out_shape=jax.ShapeDtypeStruct((_NP, _D), jnp.float32),
)


def kernel(X, edge_index, W_enc, b_enc, W_gcn1, b_gcn1, W_gcn2, b_gcn2,
           W_dec, b_dec):
    Xp = jnp.pad(X, ((0, _NP - _N), (0, 0)))
    fill = jnp.full((_NW * _CPT * _CHUNK - _E,), _NP - 1, jnp.int32)
    pad_chunk = jnp.full((_NW, 1, _CHUNK), _NP - 1, jnp.int32)

    def _chunked(idx):
        body = jnp.concatenate([idx, fill]).reshape(_NW, _CPT, _CHUNK)
        return jnp.concatenate([body, pad_chunk], axis=1)

    srcs = _chunked(edge_index[0])
    dsts = _chunked(edge_index[1])

    ones_rows = jnp.ones((_CHUNK, _D), jnp.float32)
    deg_parts = _deg_kernel(dsts, ones_rows)
    xw1 = _k0(Xp, W_enc.T, b_enc.reshape(1, _D), W_gcn1.T)
    z1, y = _k1(xw1, deg_parts)
    acc1 = _edge_sum_kernel(z1, srcs, dsts)
    z2 = _k2(acc1, z1, y, b_gcn1.reshape(1, _D), W_gcn2.T)
    acc2 = _edge_sum_kernel(z2, srcs, dsts)
    out = _k3(acc2, z2, y, b_gcn2.reshape(1, _D), W_dec.T, b_dec.reshape(1, _D))
    return out[:_N]
